# Initial kernel scaffold; baseline (speedup 1.0000x reference)
import jax
import jax.numpy as jnp
from jax.experimental import pallas as pl


def kernel(qs, types, id_table, que_table, que_W, que_b, ana_table, ana_W, ana_b, type_table):
    raise NotImplementedError("write your pallas kernel here")


# speedup vs baseline: 2.1014x; 2.1014x over previous
import functools

import jax
import jax.numpy as jnp
from jax import lax
from jax.experimental import pallas as pl
from jax.experimental.pallas import tpu as pltpu
from jax.experimental.pallas import tpu_sc as plsc


def _proj_body(t_ref, w_ref, b_ref, o_ref):
    o_ref[...] = (
        jnp.dot(t_ref[...], w_ref[...], preferred_element_type=jnp.float32)
        + b_ref[...]
    )


def _project(table, w, b):
    r, k = table.shape
    e = w.shape[1]
    blk = 2000
    assert r % blk == 0
    return pl.pallas_call(
        _proj_body,
        grid=(r // blk,),
        in_specs=[
            pl.BlockSpec((blk, k), lambda i: (i, 0)),
            pl.BlockSpec((k, e), lambda i: (0, 0)),
            pl.BlockSpec((1, e), lambda i: (0, 0)),
        ],
        out_specs=pl.BlockSpec((blk, e), lambda i: (i, 0)),
        out_shape=jax.ShapeDtypeStruct((r, e), jnp.float32),
    )(table, w, b.reshape(1, e))


_CHUNK = 256
_STREAM = 128


def _gather_all(qs_flat, types_flat, id_table, pq, pa, type_table):
    n = qs_flat.shape[0]
    e = id_table.shape[1]
    info = plsc.get_sparse_core_info()
    nc, ns = info.num_cores, info.num_subcores
    nw = nc * ns
    assert n % (nw * _CHUNK) == 0
    span = n // nw
    nchunks = span // _CHUNK

    mesh = plsc.VectorSubcoreMesh(core_axis_name="c", subcore_axis_name="s")
    out = jax.ShapeDtypeStruct((n, e), jnp.float32)

    @functools.partial(
        pl.kernel,
        out_type=(out, out, out, out),
        mesh=mesh,
        compiler_params=pltpu.CompilerParams(use_tc_tiling_on_sc=False),
        scratch_types=[
            pltpu.VMEM((_CHUNK,), jnp.int32),
            pltpu.VMEM((_CHUNK,), jnp.int32),
            pltpu.VMEM((_CHUNK, e), jnp.float32),
            pltpu.VMEM((_CHUNK, e), jnp.float32),
            pltpu.VMEM((_CHUNK, e), jnp.float32),
            pltpu.VMEM((_CHUNK, e), jnp.float32),
            pltpu.SemaphoreType.DMA,
            pltpu.SemaphoreType.DMA,
        ],
    )
    def gather_kernel(
        qs_hbm,
        ty_hbm,
        idt_hbm,
        pq_hbm,
        pa_hbm,
        tt_hbm,
        o_id,
        o_q,
        o_a,
        o_t,
        idx_v,
        tidx_v,
        b_id,
        b_q,
        b_a,
        b_t,
        gsem,
        wsem,
    ):
        wid = lax.axis_index("s") * nc + lax.axis_index("c")
        base = wid * span

        def chunk(ci, carry):
            off = base + ci * _CHUNK
            pltpu.sync_copy(qs_hbm.at[pl.ds(off, _CHUNK)], idx_v)
            pltpu.sync_copy(ty_hbm.at[pl.ds(off, _CHUNK)], tidx_v)
            cps = []
            for j in range(_CHUNK // _STREAM):
                sl = pl.ds(j * _STREAM, _STREAM)
                cps.append(pltpu.async_copy(idt_hbm.at[idx_v.at[sl]], b_id.at[sl], gsem))
                cps.append(pltpu.async_copy(pq_hbm.at[idx_v.at[sl]], b_q.at[sl], gsem))
                cps.append(pltpu.async_copy(pa_hbm.at[idx_v.at[sl]], b_a.at[sl], gsem))
                cps.append(pltpu.async_copy(tt_hbm.at[tidx_v.at[sl]], b_t.at[sl], gsem))
            for cp in cps:
                cp.wait()
            ws = [
                pltpu.async_copy(b_id, o_id.at[pl.ds(off, _CHUNK)], wsem),
                pltpu.async_copy(b_q, o_q.at[pl.ds(off, _CHUNK)], wsem),
                pltpu.async_copy(b_a, o_a.at[pl.ds(off, _CHUNK)], wsem),
                pltpu.async_copy(b_t, o_t.at[pl.ds(off, _CHUNK)], wsem),
            ]
            for cp in ws:
                cp.wait()
            return carry

        lax.fori_loop(0, nchunks, chunk, 0)

    return gather_kernel(qs_flat, types_flat, id_table, pq, pa, type_table)


def kernel(qs, types, id_table, que_table, que_W, que_b, ana_table, ana_W, ana_b, type_table):
    b, l = qs.shape
    e = id_table.shape[1]
    n = b * l
    pq = _project(que_table, que_W, que_b)
    pa = _project(ana_table, ana_W, ana_b)
    qid, cont, ana, typ = _gather_all(
        qs.reshape(n), types.reshape(n), id_table, pq, pa, type_table
    )
    return (
        qid.reshape(b, l, e),
        cont.reshape(b, l, e),
        ana.reshape(b, l, e),
        typ.reshape(b, l, e),
    )


# speedup vs baseline: 16.2578x; 7.7365x over previous
import functools

import jax
import jax.numpy as jnp
from jax import lax
from jax.experimental import pallas as pl
from jax.experimental.pallas import tpu as pltpu
from jax.experimental.pallas import tpu_sc as plsc


def _proj_body(t_ref, w_ref, b_ref, o_ref):
    o_ref[...] = (
        jnp.dot(t_ref[...], w_ref[...], preferred_element_type=jnp.float32)
        + b_ref[...]
    )


def _project(table, w, b):
    r, k = table.shape
    e = w.shape[1]
    blk = 2000
    assert r % blk == 0
    return pl.pallas_call(
        _proj_body,
        grid=(r // blk,),
        in_specs=[
            pl.BlockSpec((blk, k), lambda i: (i, 0)),
            pl.BlockSpec((k, e), lambda i: (0, 0)),
            pl.BlockSpec((1, e), lambda i: (0, 0)),
        ],
        out_specs=pl.BlockSpec((blk, e), lambda i: (i, 0)),
        out_shape=jax.ShapeDtypeStruct((r, e), jnp.float32),
    )(table, w, b.reshape(1, e))


def _type_body(t_ref, tt_ref, o_ref):
    t = t_ref[...]
    r0 = tt_ref[0]
    r1 = tt_ref[1]
    o_ref[...] = jnp.where(t[:, :, None] == 0, r0[None, None, :], r1[None, None, :])


def _type_emb(types, type_table):
    b, l = types.shape
    e = type_table.shape[1]
    bt = 256
    assert b % bt == 0
    return pl.pallas_call(
        _type_body,
        grid=(b // bt,),
        in_specs=[
            pl.BlockSpec((bt, l), lambda i: (i, 0)),
            pl.BlockSpec((2, e), lambda i: (0, 0)),
        ],
        out_specs=pl.BlockSpec((bt, l, e), lambda i: (i, 0, 0)),
        out_shape=jax.ShapeDtypeStruct((b, l, e), jnp.float32),
    )(types, type_table)


_CHUNK = 256
_STREAM = 128


def _gather3(qs_flat, id_table, pq, pa):
    n = qs_flat.shape[0]
    e = id_table.shape[1]
    info = plsc.get_sparse_core_info()
    nc, ns = info.num_cores, info.num_subcores
    nw = nc * ns
    assert n % (nw * 2 * _CHUNK) == 0
    span = n // nw
    nchunks = span // _CHUNK

    mesh = plsc.VectorSubcoreMesh(core_axis_name="c", subcore_axis_name="s")
    out = jax.ShapeDtypeStruct((n, e), jnp.float32)
    buf = pltpu.VMEM((_CHUNK, e), jnp.float32)

    @functools.partial(
        pl.kernel,
        out_type=(out, out, out),
        mesh=mesh,
        compiler_params=pltpu.CompilerParams(use_tc_tiling_on_sc=False),
        scratch_types=[
            pltpu.VMEM((span,), jnp.int32),
            (buf, buf, buf),
            (buf, buf, buf),
            pltpu.SemaphoreType.DMA,
            pltpu.SemaphoreType.DMA,
            pltpu.SemaphoreType.DMA,
            pltpu.SemaphoreType.DMA,
        ],
    )
    def gather_kernel(
        qs_hbm,
        idt_hbm,
        pq_hbm,
        pa_hbm,
        o_id,
        o_q,
        o_a,
        idx_v,
        bufs0,
        bufs1,
        gsem0,
        gsem1,
        wsem0,
        wsem1,
    ):
        wid = lax.axis_index("s") * nc + lax.axis_index("c")
        base = wid * span
        tabs = (idt_hbm, pq_hbm, pa_hbm)
        outs = (o_id, o_q, o_a)

        pltpu.sync_copy(qs_hbm.at[pl.ds(base, span)], idx_v)

        def fire_gathers(ci, bufs, sem):
            cps = []
            for j in range(_CHUNK // _STREAM):
                sl = pl.ds(ci * _CHUNK + j * _STREAM, _STREAM)
                dsl = pl.ds(j * _STREAM, _STREAM)
                for tab, bf in zip(tabs, bufs):
                    cps.append(pltpu.async_copy(tab.at[idx_v.at[sl]], bf.at[dsl], sem))
            return cps

        def fire_writes(ci, bufs, sem):
            off = base + ci * _CHUNK
            for bf, o in zip(bufs, outs):
                pltpu.async_copy(bf, o.at[pl.ds(off, _CHUNK)], sem)

        def wait_writes(ci, bufs, sem):
            off = base + ci * _CHUNK
            for bf, o in zip(bufs, outs):
                pltpu.make_async_copy(bf, o.at[pl.ds(off, _CHUNK)], sem).wait()

        def body(k, carry):
            a = 2 * k
            b = a + 1

            @pl.when(k >= 1)
            def _():
                wait_writes(a - 2, bufs0, wsem0)

            ga = fire_gathers(a, bufs0, gsem0)

            @pl.when(k >= 1)
            def _():
                wait_writes(b - 2, bufs1, wsem1)

            gb = fire_gathers(b, bufs1, gsem1)

            for cp in ga:
                cp.wait()
            fire_writes(a, bufs0, wsem0)
            for cp in gb:
                cp.wait()
            fire_writes(b, bufs1, wsem1)
            return carry

        lax.fori_loop(0, nchunks // 2, body, 0)
        wait_writes(nchunks - 2, bufs0, wsem0)
        wait_writes(nchunks - 1, bufs1, wsem1)

    return gather_kernel(qs_flat, id_table, pq, pa)


def kernel(qs, types, id_table, que_table, que_W, que_b, ana_table, ana_W, ana_b, type_table):
    b, l = qs.shape
    e = id_table.shape[1]
    n = b * l
    pq = _project(que_table, que_W, que_b)
    pa = _project(ana_table, ana_W, ana_b)
    typ = _type_emb(types, type_table)
    qid, cont, ana = _gather3(qs.reshape(n), id_table, pq, pa)
    return (
        qid.reshape(b, l, e),
        cont.reshape(b, l, e),
        ana.reshape(b, l, e),
        typ,
    )


# speedup vs baseline: 16.2625x; 1.0003x over previous
import functools

import jax
import jax.numpy as jnp
from jax import lax
from jax.experimental import pallas as pl
from jax.experimental.pallas import tpu as pltpu
from jax.experimental.pallas import tpu_sc as plsc


def _proj_body(t_ref, w_ref, b_ref, o_ref):
    o_ref[...] = (
        jnp.dot(t_ref[...], w_ref[...], preferred_element_type=jnp.float32)
        + b_ref[...]
    )


def _project(table, w, b):
    r, k = table.shape
    e = w.shape[1]
    blk = 2000
    assert r % blk == 0
    return pl.pallas_call(
        _proj_body,
        grid=(r // blk,),
        in_specs=[
            pl.BlockSpec((blk, k), lambda i: (i, 0)),
            pl.BlockSpec((k, e), lambda i: (0, 0)),
            pl.BlockSpec((1, e), lambda i: (0, 0)),
        ],
        out_specs=pl.BlockSpec((blk, e), lambda i: (i, 0)),
        out_shape=jax.ShapeDtypeStruct((r, e), jnp.float32),
    )(table, w, b.reshape(1, e))


def _type_body(t_ref, tt_ref, o_ref):
    t = t_ref[...]
    r0 = tt_ref[0]
    r1 = tt_ref[1]
    o_ref[...] = jnp.where(t[:, :, None] == 0, r0[None, None, :], r1[None, None, :])


def _type_emb(types, type_table):
    b, l = types.shape
    e = type_table.shape[1]
    bt = 256
    assert b % bt == 0
    return pl.pallas_call(
        _type_body,
        grid=(b // bt,),
        in_specs=[
            pl.BlockSpec((bt, l), lambda i: (i, 0)),
            pl.BlockSpec((2, e), lambda i: (0, 0)),
        ],
        out_specs=pl.BlockSpec((bt, l, e), lambda i: (i, 0, 0)),
        out_shape=jax.ShapeDtypeStruct((b, l, e), jnp.float32),
    )(types, type_table)


_CHUNK = 256
_STREAM = 256


def _gather3(qs_flat, id_table, pq, pa):
    n = qs_flat.shape[0]
    e = id_table.shape[1]
    info = plsc.get_sparse_core_info()
    nc, ns = info.num_cores, info.num_subcores
    nw = nc * ns
    assert n % (nw * 2 * _CHUNK) == 0
    span = n // nw
    nchunks = span // _CHUNK

    mesh = plsc.VectorSubcoreMesh(core_axis_name="c", subcore_axis_name="s")
    out = jax.ShapeDtypeStruct((n, e), jnp.float32)
    buf = pltpu.VMEM((_CHUNK, e), jnp.float32)

    @functools.partial(
        pl.kernel,
        out_type=(out, out, out),
        mesh=mesh,
        compiler_params=pltpu.CompilerParams(use_tc_tiling_on_sc=False),
        scratch_types=[
            pltpu.VMEM((span,), jnp.int32),
            (buf, buf, buf),
            (buf, buf, buf),
            pltpu.SemaphoreType.DMA,
            pltpu.SemaphoreType.DMA,
            pltpu.SemaphoreType.DMA,
            pltpu.SemaphoreType.DMA,
        ],
    )
    def gather_kernel(
        qs_hbm,
        idt_hbm,
        pq_hbm,
        pa_hbm,
        o_id,
        o_q,
        o_a,
        idx_v,
        bufs0,
        bufs1,
        gsem0,
        gsem1,
        wsem0,
        wsem1,
    ):
        wid = lax.axis_index("s") * nc + lax.axis_index("c")
        base = wid * span
        tabs = (idt_hbm, pq_hbm, pa_hbm)
        outs = (o_id, o_q, o_a)

        pltpu.sync_copy(qs_hbm.at[pl.ds(base, span)], idx_v)

        def fire_gathers(ci, bufs, sem):
            cps = []
            for j in range(_CHUNK // _STREAM):
                sl = pl.ds(ci * _CHUNK + j * _STREAM, _STREAM)
                dsl = pl.ds(j * _STREAM, _STREAM)
                for tab, bf in zip(tabs, bufs):
                    cps.append(pltpu.async_copy(tab.at[idx_v.at[sl]], bf.at[dsl], sem))
            return cps

        def fire_writes(ci, bufs, sem):
            off = base + ci * _CHUNK
            for bf, o in zip(bufs, outs):
                pltpu.async_copy(bf, o.at[pl.ds(off, _CHUNK)], sem)

        def wait_writes(ci, bufs, sem):
            off = base + ci * _CHUNK
            for bf, o in zip(bufs, outs):
                pltpu.make_async_copy(bf, o.at[pl.ds(off, _CHUNK)], sem).wait()

        def body(k, carry):
            a = 2 * k
            b = a + 1

            @pl.when(k >= 1)
            def _():
                wait_writes(a - 2, bufs0, wsem0)

            ga = fire_gathers(a, bufs0, gsem0)

            @pl.when(k >= 1)
            def _():
                wait_writes(b - 2, bufs1, wsem1)

            gb = fire_gathers(b, bufs1, gsem1)

            for cp in ga:
                cp.wait()
            fire_writes(a, bufs0, wsem0)
            for cp in gb:
                cp.wait()
            fire_writes(b, bufs1, wsem1)
            return carry

        lax.fori_loop(0, nchunks // 2, body, 0)
        wait_writes(nchunks - 2, bufs0, wsem0)
        wait_writes(nchunks - 1, bufs1, wsem1)

    return gather_kernel(qs_flat, id_table, pq, pa)


def kernel(qs, types, id_table, que_table, que_W, que_b, ana_table, ana_W, ana_b, type_table):
    b, l = qs.shape
    e = id_table.shape[1]
    n = b * l
    pq = _project(que_table, que_W, que_b)
    pa = _project(ana_table, ana_W, ana_b)
    typ = _type_emb(types, type_table)
    qid, cont, ana = _gather3(qs.reshape(n), id_table, pq, pa)
    return (
        qid.reshape(b, l, e),
        cont.reshape(b, l, e),
        ana.reshape(b, l, e),
        typ,
    )


# speedup vs baseline: 16.7528x; 1.0301x over previous
import functools

import jax
import jax.numpy as jnp
from jax import lax
from jax.experimental import pallas as pl
from jax.experimental.pallas import tpu as pltpu
from jax.experimental.pallas import tpu_sc as plsc


def _proj_body(t_ref, w_ref, b_ref, o_ref):
    o_ref[...] = (
        jnp.dot(t_ref[...], w_ref[...], preferred_element_type=jnp.float32)
        + b_ref[...]
    )


def _project(table, w, b):
    r, k = table.shape
    e = w.shape[1]
    blk = 2000
    assert r % blk == 0
    return pl.pallas_call(
        _proj_body,
        grid=(r // blk,),
        in_specs=[
            pl.BlockSpec((blk, k), lambda i: (i, 0)),
            pl.BlockSpec((k, e), lambda i: (0, 0)),
            pl.BlockSpec((1, e), lambda i: (0, 0)),
        ],
        out_specs=pl.BlockSpec((blk, e), lambda i: (i, 0)),
        out_shape=jax.ShapeDtypeStruct((r, e), jnp.float32),
    )(table, w, b.reshape(1, e))


def _type_body(t_ref, tt_ref, o_ref):
    t = t_ref[...]
    r0 = tt_ref[0]
    r1 = tt_ref[1]
    o_ref[...] = jnp.where(t[:, :, None] == 0, r0[None, None, :], r1[None, None, :])


def _type_emb(types, type_table):
    b, l = types.shape
    e = type_table.shape[1]
    bt = 256
    assert b % bt == 0
    return pl.pallas_call(
        _type_body,
        grid=(b // bt,),
        in_specs=[
            pl.BlockSpec((bt, l), lambda i: (i, 0)),
            pl.BlockSpec((2, e), lambda i: (0, 0)),
        ],
        out_specs=pl.BlockSpec((bt, l, e), lambda i: (i, 0, 0)),
        out_shape=jax.ShapeDtypeStruct((b, l, e), jnp.float32),
    )(types, type_table)


_CHUNK = 256
_STREAM = 256


def _gather_tabs(qs_flat, *tables):
    n = qs_flat.shape[0]
    e = tables[0].shape[1]
    info = plsc.get_sparse_core_info()
    nc, ns = info.num_cores, info.num_subcores
    nw = nc * ns
    assert n % (nw * 2 * _CHUNK) == 0
    span = n // nw
    nchunks = span // _CHUNK

    mesh = plsc.VectorSubcoreMesh(core_axis_name="c", subcore_axis_name="s")
    nt = len(tables)
    out = jax.ShapeDtypeStruct((n, e), jnp.float32)
    buf = pltpu.VMEM((_CHUNK, e), jnp.float32)

    @functools.partial(
        pl.kernel,
        out_type=(out,) * nt,
        mesh=mesh,
        compiler_params=pltpu.CompilerParams(use_tc_tiling_on_sc=False),
        scratch_types=[
            pltpu.VMEM((span,), jnp.int32),
            (buf,) * nt,
            (buf,) * nt,
            pltpu.SemaphoreType.DMA,
            pltpu.SemaphoreType.DMA,
            pltpu.SemaphoreType.DMA,
            pltpu.SemaphoreType.DMA,
        ],
    )
    def gather_kernel(qs_hbm, *rest):
        tabs = rest[:nt]
        outs = rest[nt : 2 * nt]
        idx_v, bufs0, bufs1, gsem0, gsem1, wsem0, wsem1 = rest[2 * nt :]
        wid = lax.axis_index("s") * nc + lax.axis_index("c")
        base = wid * span

        pltpu.sync_copy(qs_hbm.at[pl.ds(base, span)], idx_v)

        def fire_gathers(ci, bufs, sem):
            cps = []
            for j in range(_CHUNK // _STREAM):
                sl = pl.ds(ci * _CHUNK + j * _STREAM, _STREAM)
                dsl = pl.ds(j * _STREAM, _STREAM)
                for tab, bf in zip(tabs, bufs):
                    cps.append(pltpu.async_copy(tab.at[idx_v.at[sl]], bf.at[dsl], sem))
            return cps

        def fire_writes(ci, bufs, sem):
            off = base + ci * _CHUNK
            for bf, o in zip(bufs, outs):
                pltpu.async_copy(bf, o.at[pl.ds(off, _CHUNK)], sem)

        def wait_writes(ci, bufs, sem):
            off = base + ci * _CHUNK
            for bf, o in zip(bufs, outs):
                pltpu.make_async_copy(bf, o.at[pl.ds(off, _CHUNK)], sem).wait()

        def body(k, carry):
            a = 2 * k
            b = a + 1

            @pl.when(k >= 1)
            def _():
                wait_writes(a - 2, bufs0, wsem0)

            ga = fire_gathers(a, bufs0, gsem0)

            @pl.when(k >= 1)
            def _():
                wait_writes(b - 2, bufs1, wsem1)

            gb = fire_gathers(b, bufs1, gsem1)

            for cp in ga:
                cp.wait()
            fire_writes(a, bufs0, wsem0)
            for cp in gb:
                cp.wait()
            fire_writes(b, bufs1, wsem1)
            return carry

        lax.fori_loop(0, nchunks // 2, body, 0)
        wait_writes(nchunks - 2, bufs0, wsem0)
        wait_writes(nchunks - 1, bufs1, wsem1)

    return gather_kernel(qs_flat, *tables)


def kernel(qs, types, id_table, que_table, que_W, que_b, ana_table, ana_W, ana_b, type_table):
    b, l = qs.shape
    e = id_table.shape[1]
    n = b * l
    qs_flat = qs.reshape(n)
    (qid,) = _gather_tabs(qs_flat, id_table)
    pq = _project(que_table, que_W, que_b)
    pa = _project(ana_table, ana_W, ana_b)
    typ = _type_emb(types, type_table)
    cont, ana = _gather_tabs(qs_flat, pq, pa)
    return (
        qid.reshape(b, l, e),
        cont.reshape(b, l, e),
        ana.reshape(b, l, e),
        typ,
    )
